# baseline (speedup 1.0000x reference)
import functools

import jax
import jax.numpy as jnp
from jax.experimental import pallas as pl
import jax.experimental.pallas.tpu as pltpu


def _vq_body(zt_ref, cb_ref, o_ref, cbm2_ref, e2_ref, *, bk, n_kc, sl_rows):
    b = pl.program_id(0)
    k_total, c_dim = cb_ref.shape
    w = zt_ref.shape[2]

    @pl.when(b == 0)
    def _build():
        for kc in range(n_kc):
            rows = slice(kc * bk, (kc + 1) * bk)
            cb = cb_ref[rows, :]
            cbm2_ref[rows, :] = cb * -2.0
            e2_ref[rows, :] = jnp.sum(cb * cb, axis=1, keepdims=True)

    lat = zt_ref[0]
    zz = jnp.sum(lat * lat, axis=0, keepdims=True)

    n_slices = bk // sl_rows
    rm = jnp.full((sl_rows, w), jnp.inf, jnp.float32)
    si = jnp.zeros((sl_rows, w), jnp.float32)

    for kc in range(n_kc):
        sm2 = jax.lax.dot_general(
            cbm2_ref[kc * bk:(kc + 1) * bk, :], lat,
            (((1,), (0,)), ((), ())),
            preferred_element_type=jnp.float32)
        for sl in range(n_slices):
            rows = slice(sl * sl_rows, (sl + 1) * sl_rows)
            t = zz + e2_ref[kc * bk + sl * sl_rows:
                            kc * bk + (sl + 1) * sl_rows, :]
            d = t + sm2[rows, :]
            gs = jnp.float32(kc * n_slices + sl)
            upd = d < rm
            si = jnp.where(upd, gs, si)
            rm = jnp.where(upd, d, rm)

    pos = jax.lax.broadcasted_iota(jnp.int32, (sl_rows, w), 0).astype(jnp.float32)
    rows_g = si * jnp.float32(sl_rows) + pos
    gmin = jnp.min(rm, axis=0, keepdims=True)
    cand = jnp.where(rm == gmin, rows_g, jnp.inf)
    best = jnp.min(cand, axis=0, keepdims=True)
    o_ref[0] = best.astype(jnp.int32)


def kernel(latent, codebook):
    B, C, H, W = latent.shape
    K, _ = codebook.shape
    n_tok = H * W
    zt = latent.reshape(B, C, n_tok)

    BK = 1024
    n_kc = K // BK
    SL = 32

    out = pl.pallas_call(
        functools.partial(_vq_body, bk=BK, n_kc=n_kc, sl_rows=SL),
        grid=(B,),
        in_specs=[
            pl.BlockSpec((1, C, n_tok), lambda b: (b, 0, 0)),
            pl.BlockSpec((K, C), lambda b: (0, 0)),
        ],
        out_specs=pl.BlockSpec((1, 1, n_tok), lambda b: (b, 0, 0)),
        out_shape=jax.ShapeDtypeStruct((B, 1, n_tok), jnp.int32),
        scratch_shapes=[
            pltpu.VMEM((K, C), jnp.float32),
            pltpu.VMEM((K, 1), jnp.float32),
        ],
        compiler_params=pltpu.CompilerParams(
            dimension_semantics=("arbitrary",),
        ),
    )(zt, codebook)
    return out.reshape(B, n_tok)


# speedup vs baseline: 1.0326x; 1.0326x over previous
import functools

import jax
import jax.numpy as jnp
from jax.experimental import pallas as pl
import jax.experimental.pallas.tpu as pltpu


def _vq_body(zt_ref, cb_ref, o_ref, cbm2_ref, e2_ref, *, bk, n_kc, sl_rows):
    b = pl.program_id(0)
    w = zt_ref.shape[2]

    @pl.when(b == 0)
    def _build():
        for kc in range(n_kc):
            rows = slice(kc * bk, (kc + 1) * bk)
            cb = cb_ref[rows, :]
            cbm2_ref[rows, :] = cb * -2.0
            e2 = jnp.sum(cb * cb, axis=1, keepdims=True)
            e2_ref[rows, :] = jnp.broadcast_to(e2, (bk, w))

    lat = zt_ref[0]
    zz = jnp.sum(lat * lat, axis=0, keepdims=True)

    n_slices = bk // sl_rows
    inf = jnp.full((sl_rows, w), jnp.inf, jnp.float32)
    zero = jnp.zeros((sl_rows, w), jnp.float32)
    rm = [inf, inf]
    si = [zero, zero]

    for kc in range(n_kc):
        sm2 = jax.lax.dot_general(
            cbm2_ref[kc * bk:(kc + 1) * bk, :], lat,
            (((1,), (0,)), ((), ())),
            preferred_element_type=jnp.float32)
        for sl in range(n_slices):
            rows = slice(sl * sl_rows, (sl + 1) * sl_rows)
            t = zz + e2_ref[kc * bk + sl * sl_rows:
                            kc * bk + (sl + 1) * sl_rows, :]
            d = t + sm2[rows, :]
            gs = jnp.float32(kc * n_slices + sl)
            p = sl & 1
            upd = d < rm[p]
            si[p] = jnp.where(upd, gs, si[p])
            rm[p] = jnp.minimum(rm[p], d)

    swap = rm[1] < rm[0]
    rm_m = jnp.minimum(rm[0], rm[1])
    si_m = jnp.where(swap, si[1], si[0])
    tie = rm[0] == rm[1]
    si_m = jnp.where(tie, jnp.minimum(si[0], si[1]), si_m)

    pos = jax.lax.broadcasted_iota(jnp.int32, (sl_rows, w), 0).astype(jnp.float32)
    rows_g = si_m * jnp.float32(sl_rows) + pos
    gmin = jnp.min(rm_m, axis=0, keepdims=True)
    cand = jnp.where(rm_m == gmin, rows_g, jnp.inf)
    best = jnp.min(cand, axis=0, keepdims=True)
    o_ref[0] = best.astype(jnp.int32)


def kernel(latent, codebook):
    B, C, H, W = latent.shape
    K, _ = codebook.shape
    n_tok = H * W
    zt = latent.reshape(B, C, n_tok)

    BK = 1024
    n_kc = K // BK
    SL = 32

    out = pl.pallas_call(
        functools.partial(_vq_body, bk=BK, n_kc=n_kc, sl_rows=SL),
        grid=(B,),
        in_specs=[
            pl.BlockSpec((1, C, n_tok), lambda b: (b, 0, 0)),
            pl.BlockSpec((K, C), lambda b: (0, 0)),
        ],
        out_specs=pl.BlockSpec((1, 1, n_tok), lambda b: (b, 0, 0)),
        out_shape=jax.ShapeDtypeStruct((B, 1, n_tok), jnp.int32),
        scratch_shapes=[
            pltpu.VMEM((K, C), jnp.float32),
            pltpu.VMEM((K, n_tok), jnp.float32),
        ],
        compiler_params=pltpu.CompilerParams(
            dimension_semantics=("arbitrary",),
        ),
    )(zt, codebook)
    return out.reshape(B, n_tok)


# speedup vs baseline: 1.1515x; 1.1151x over previous
import functools

import jax
import jax.numpy as jnp
from jax.experimental import pallas as pl
import jax.experimental.pallas.tpu as pltpu


def _vq_body(zt_ref, cb_ref, o_ref, cbm2_ref, e2_ref, *, bk, n_kc, sl_rows, dot_rows):
    b = pl.program_id(0)
    k_total, c_dim = cb_ref.shape
    w = zt_ref.shape[2]

    @pl.when(b == 0)
    def _build():
        for kc in range(n_kc):
            rows = slice(kc * bk, (kc + 1) * bk)
            cb = cb_ref[rows, :]
            cbm2_ref[rows, :] = cb * -2.0
            e2 = jnp.sum(cb * cb, axis=1, keepdims=True)
            e2_ref[rows, :] = jnp.broadcast_to(e2, (bk, w))

    lat = zt_ref[0]
    zz = jnp.sum(lat * lat, axis=0, keepdims=True)

    rm = jnp.full((sl_rows, w), jnp.inf, jnp.float32)
    si = jnp.zeros((sl_rows, w), jnp.float32)

    k_total = cb_ref.shape[0]
    n_dots = k_total // dot_rows
    per_dot = dot_rows // sl_rows
    for kd in range(n_dots):
        sm2 = jax.lax.dot_general(
            cbm2_ref[kd * dot_rows:(kd + 1) * dot_rows, :], lat,
            (((1,), (0,)), ((), ())),
            preferred_element_type=jnp.float32)
        for sl in range(per_dot):
            rows = slice(sl * sl_rows, (sl + 1) * sl_rows)
            t = zz + e2_ref[kd * dot_rows + sl * sl_rows:
                            kd * dot_rows + (sl + 1) * sl_rows, :]
            d = t + sm2[rows, :]
            gs = jnp.float32(kd * per_dot + sl)
            upd = d < rm
            si = jnp.where(upd, gs, si)
            rm = jnp.where(upd, d, rm)

    pos = jax.lax.broadcasted_iota(jnp.int32, (sl_rows, w), 0).astype(jnp.float32)
    rows_g = si * jnp.float32(sl_rows) + pos
    gmin = jnp.min(rm, axis=0, keepdims=True)
    cand = jnp.where(rm == gmin, rows_g, jnp.inf)
    best = jnp.min(cand, axis=0, keepdims=True)
    o_ref[0] = best.astype(jnp.int32)


def kernel(latent, codebook):
    B, C, H, W = latent.shape
    K, _ = codebook.shape
    n_tok = H * W
    zt = latent.reshape(B, C, n_tok)

    BK = 1024
    n_kc = K // BK
    SL = 32
    DR = 128

    out = pl.pallas_call(
        functools.partial(_vq_body, bk=BK, n_kc=n_kc, sl_rows=SL, dot_rows=DR),
        grid=(B,),
        in_specs=[
            pl.BlockSpec((1, C, n_tok), lambda b: (b, 0, 0)),
            pl.BlockSpec((K, C), lambda b: (0, 0)),
        ],
        out_specs=pl.BlockSpec((1, 1, n_tok), lambda b: (b, 0, 0)),
        out_shape=jax.ShapeDtypeStruct((B, 1, n_tok), jnp.int32),
        scratch_shapes=[
            pltpu.VMEM((K, C), jnp.float32),
            pltpu.VMEM((K, n_tok), jnp.float32),
        ],
        compiler_params=pltpu.CompilerParams(
            dimension_semantics=("arbitrary",),
        ),
    )(zt, codebook)
    return out.reshape(B, n_tok)
